# baseline (speedup 1.0000x reference)
import jax
import jax.numpy as jnp
from jax import lax
from jax.experimental import pallas as pl
from jax.experimental.pallas import tpu as pltpu
from jax.experimental.pallas import tpu_sc as plsc

N_NODES = 1_000_000
NUM_RELS = 32
D = 64
B = 16384
K = 10
L = 16
NC, NS = 2, 16
NW = NC * NS
BPW = B // NW
SUB = 128
NSUB = BPW // SUB
SL = 62512
SL2 = SL // 2
EB = 16
MAXE = 2 * SUB


def _iota16():
    return lax.iota(jnp.int32, L)


def _body(node2, rel2, head_idx, tail_idx, rel_t, deg, sim_f, nid_f,
          score_hbm, owner_sh, head_full, owner_sl, relv_v, hidx_v, tidx_v,
          hidx2_v, tidx2_v, rl_v, wh_v, wt_v, wtc_v, relw_v, went_v, qref_v,
          eidx_v, kidx_v, sims_e, deg_e, nflat_v, parn_v, neigh_v, hn_v,
          tn_v, scores_v, sem):
    cid = lax.axis_index("c")
    sid = lax.axis_index("s")
    wid = cid * NS + sid
    pltpu.sync_copy(head_idx, head_full)
    pltpu.sync_copy(rel2, relv_v)

    for p in range(2):
        base = sid * SL + p * SL2

        def oscan(g, _, base=base):
            hv = head_full[pl.ds(g * L, L)]
            j = g * L + _iota16()
            local = hv - base
            m = (local >= 0) & (local < SL2)
            lc = jnp.minimum(jnp.maximum(local, 0), SL2 - 1)
            plsc.store_scatter(owner_sl, [lc], j, mask=m)
            g2 = plsc.load_gather(owner_sl, [lc], mask=m)
            plsc.store_scatter(owner_sl, [lc], j, mask=m & (g2 < j))
            g3 = plsc.load_gather(owner_sl, [lc], mask=m)
            plsc.store_scatter(owner_sl, [lc], j, mask=m & (g3 < j))
            return 0
        lax.fori_loop(0, B // L, oscan, 0)
        pltpu.sync_copy(owner_sl, owner_sh.at[pl.ds(base, SL2)])
    plsc.subcore_barrier()

    def sub(s, _):
        rowbase = wid * BPW + s * SUB
        pltpu.sync_copy(head_idx.at[pl.ds(rowbase, SUB)], hidx_v)
        pltpu.sync_copy(tail_idx.at[pl.ds(rowbase, SUB)], tidx_v)
        pltpu.sync_copy(rel_t.at[pl.ds(rowbase, SUB)], rl_v)
        for g in range(SUB // L):
            sl_ = pl.ds(g * L, L)
            hidx2_v[sl_] = hidx_v[sl_] >> 1
            tidx2_v[sl_] = tidx_v[sl_] >> 1
        c1 = pltpu.async_copy(owner_sh.at[hidx_v], wh_v, sem)
        c2 = pltpu.async_copy(owner_sh.at[tidx_v], wt_v, sem)
        c3 = pltpu.async_copy(node2.at[hidx2_v], hn_v, sem)
        c4 = pltpu.async_copy(node2.at[tidx2_v], tn_v, sem)
        c1.wait()
        c2.wait()
        c3.wait()
        c4.wait()
        for g in range(SUB // L):
            sl_ = pl.ds(g * L, L)
            wt = wt_v[sl_]
            wtc_v[sl_] = jnp.minimum(jnp.maximum(wt, 0), B - 1)
        c5 = pltpu.async_copy(rel_t.at[wh_v], relw_v, sem)
        c6 = pltpu.async_copy(rel_t.at[wtc_v], wt_v, sem)
        c5.wait()
        c6.wait()

        def centry(g, cnt):
            sl_ = pl.ds(g * L, L)
            qpos = g * L + _iota16()
            mh = (relw_v[sl_] >= 2) & (relw_v[sl_] <= 4)
            pos = cnt + plsc.cumsum(mh.astype(jnp.int32)) - 1
            plsc.store_scatter(went_v, [pos], wh_v[sl_], mask=mh)
            plsc.store_scatter(qref_v, [pos], qpos, mask=mh)
            cnt = cnt + jnp.sum(mh.astype(jnp.int32))
            wt0 = plsc.load_gather(head_full, [wtc_v[sl_]])
            ok = (wt_v[sl_] >= 2) & (wt_v[sl_] <= 4) & (wt0 == tidx_v[sl_])
            pos = cnt + plsc.cumsum(ok.astype(jnp.int32)) - 1
            plsc.store_scatter(went_v, [pos], wtc_v[sl_], mask=ok)
            plsc.store_scatter(qref_v, [pos], qpos + SUB, mask=ok)
            return cnt + jnp.sum(ok.astype(jnp.int32))
        nent = lax.fori_loop(0, SUB // L, centry, jnp.int32(0))

        def eblock(b, _):
            ei = b * EB + _iota16()
            elane = ei < nent
            eic = jnp.minimum(ei, MAXE - 1)
            went = plsc.load_gather(went_v, [eic], mask=elane)
            went = jnp.where(elane, went, _iota16())
            qref = plsc.load_gather(qref_v, [eic], mask=elane)
            qref = jnp.where(elane, qref, 0)
            eidx_v[pl.ds(0, L)] = went
            for k in range(K):
                plsc.store_scatter(kidx_v, [_iota16() * K + k], went * K + k)
            ce1 = pltpu.async_copy(sim_f.at[kidx_v.at[pl.ds(0, 128)]],
                                   sims_e.at[pl.ds(0, 128)], sem)
            ce2 = pltpu.async_copy(sim_f.at[kidx_v.at[pl.ds(128, 32)]],
                                   sims_e.at[pl.ds(128, 32)], sem)
            ce3 = pltpu.async_copy(nid_f.at[kidx_v.at[pl.ds(0, 128)]],
                                   nflat_v.at[pl.ds(0, 128)], sem)
            ce4 = pltpu.async_copy(nid_f.at[kidx_v.at[pl.ds(128, 32)]],
                                   nflat_v.at[pl.ds(128, 32)], sem)
            ce5 = pltpu.async_copy(deg.at[eidx_v], deg_e, sem)
            ce1.wait()
            ce2.wait()
            ce3.wait()
            ce4.wait()
            ce5.wait()
            for j in range(EB * K // L):
                sl_ = pl.ds(j * L, L)
                v = nflat_v[sl_]
                parn_v[sl_] = (v & 1) * D
                nflat_v[sl_] = v >> 1
            cn1 = pltpu.async_copy(node2.at[nflat_v.at[pl.ds(0, 128)]],
                                   neigh_v.at[pl.ds(0, 128)], sem)
            cn2 = pltpu.async_copy(node2.at[nflat_v.at[pl.ds(128, 32)]],
                                   neigh_v.at[pl.ds(128, 32)], sem)
            dge = deg_e[pl.ds(0, L)]
            cc = 0.7 * jnp.exp(-0.7 * dge.astype(jnp.float32)) + 0.2
            cn1.wait()
            cn2.wait()
            sk = [plsc.load_gather(sims_e, [_iota16() * K + k])
                  for k in range(K)]
            pk = [plsc.load_gather(parn_v, [_iota16() * K + k])
                  for k in range(K)]
            ishead = qref < SUB
            qp = jnp.minimum(qref, SUB - 1)
            qp2 = jnp.minimum(jnp.maximum(qref - SUB, 0), SUB - 1)
            ph = (plsc.load_gather(hidx_v, [qp]) & 1) * D
            pt = (plsc.load_gather(tidx_v, [qp2]) & 1) * D
            colp = jnp.where(ishead, ph, pt)

            def dbody(d, _, sk=sk, pk=pk, cc=cc, ishead=ishead, qp=qp,
                      qp2=qp2, colp=colp, elane=elane):
                dv = jnp.full((L,), d, jnp.int32)
                acc = jnp.zeros((L,), jnp.float32)
                for k in range(K):
                    acc += sk[k] * plsc.load_gather(
                        neigh_v, [_iota16() * K + k, dv + pk[k]])
                col = dv + colp
                oh = jnp.where(ishead,
                               plsc.load_gather(hn_v, [qp, col]),
                               plsc.load_gather(tn_v, [qp2, col]))
                val = cc * acc + (1.0 - cc) * oh
                plsc.store_scatter(hn_v, [qp, col], val, mask=ishead & elane)
                plsc.store_scatter(tn_v, [qp2, col], val,
                                   mask=(~ishead) & elane)
                return 0
            lax.fori_loop(0, D, dbody, 0)
            return 0
        nblk = (nent + EB - 1) // EB
        lax.fori_loop(0, nblk, eblock, 0)

        for g in range(SUB // L):
            row = g * L + _iota16()
            rl = rl_v[pl.ds(g * L, L)]
            hcol0 = (hidx_v[pl.ds(g * L, L)] & 1) * D
            tcol0 = (tidx_v[pl.ds(g * L, L)] & 1) * D
            rrow = rl >> 1
            rcol0 = (rl & 1) * D

            def sbody(d, acc, row=row, rrow=rrow, hcol0=hcol0, tcol0=tcol0,
                      rcol0=rcol0):
                dv = jnp.full((L,), d, jnp.int32)
                h = plsc.load_gather(hn_v, [row, dv + hcol0])
                t = plsc.load_gather(tn_v, [row, dv + tcol0])
                r = plsc.load_gather(relv_v, [rrow, dv + rcol0])
                return acc + h * r * t
            acc = lax.fori_loop(0, D, sbody, jnp.zeros((L,), jnp.float32))
            scores_v[pl.ds(s * SUB + g * L, L)] = 1.0 / (1.0 + jnp.exp(-acc))
        return 0
    lax.fori_loop(0, NSUB, sub, 0)
    pltpu.sync_copy(scores_v, score_hbm.at[pl.ds(wid * BPW, BPW)])


def kernel(node_emb, rel_emb, head_indices, rel_types, tail_indices,
           sim_scores, neighbor_idx, degrees):
    head_i = head_indices.astype(jnp.int32)
    tail_i = tail_indices.astype(jnp.int32)
    rel_i = rel_types.astype(jnp.int32)
    deg_i = degrees.astype(jnp.int32)
    sim_f = sim_scores.reshape(B * K)
    nid_f = neighbor_idx.astype(jnp.int32).reshape(B * K)
    node2 = node_emb.reshape(N_NODES // 2, 2 * D)
    rel2 = rel_emb.reshape(NUM_RELS // 2, 2 * D)

    mesh = plsc.VectorSubcoreMesh(core_axis_name="c", subcore_axis_name="s")
    k = pl.kernel(
        _body,
        out_type=(jax.ShapeDtypeStruct((B,), jnp.float32),
                  jax.ShapeDtypeStruct((NS * SL,), jnp.int32)),
        mesh=mesh,
        compiler_params=pltpu.CompilerParams(
            needs_layout_passes=False, use_tc_tiling_on_sc=True),
        scratch_types=[
            pltpu.VMEM((B,), jnp.int32),
            pltpu.VMEM((SL2,), jnp.int32),
            pltpu.VMEM((NUM_RELS // 2, 2 * D), jnp.float32),
            pltpu.VMEM((SUB,), jnp.int32),
            pltpu.VMEM((SUB,), jnp.int32),
            pltpu.VMEM((SUB,), jnp.int32),
            pltpu.VMEM((SUB,), jnp.int32),
            pltpu.VMEM((SUB,), jnp.int32),
            pltpu.VMEM((SUB,), jnp.int32),
            pltpu.VMEM((SUB,), jnp.int32),
            pltpu.VMEM((SUB,), jnp.int32),
            pltpu.VMEM((SUB,), jnp.int32),
            pltpu.VMEM((MAXE,), jnp.int32),
            pltpu.VMEM((MAXE,), jnp.int32),
            pltpu.VMEM((L,), jnp.int32),
            pltpu.VMEM((EB * K,), jnp.int32),
            pltpu.VMEM((EB * K,), jnp.float32),
            pltpu.VMEM((L,), jnp.int32),
            pltpu.VMEM((EB * K,), jnp.int32),
            pltpu.VMEM((EB * K,), jnp.int32),
            pltpu.VMEM((EB * K, 2 * D), jnp.float32),
            pltpu.VMEM((SUB, 2 * D), jnp.float32),
            pltpu.VMEM((SUB, 2 * D), jnp.float32),
            pltpu.VMEM((BPW,), jnp.float32),
            pltpu.SemaphoreType.DMA,
        ],
    )
    return k(node2, rel2, head_i, tail_i, rel_i, deg_i, sim_f, nid_f)[0]
